# baseline (speedup 1.0000x reference)
import functools

import jax
import jax.numpy as jnp
from jax import lax
from jax.experimental import pallas as pl
from jax.experimental.pallas import tpu as pltpu
from jax.experimental.pallas import tpu_sc as plsc

NUM_ROWS = 16384
NUM_COLS = 26
DIM = 32
NUM_EMB = 1000000
NW = 32
CHUNK = 512
KPC = NUM_ROWS // CHUNK
N_ITEMS = NUM_COLS * KPC
IPW = N_ITEMS // NW
LANES = 16


def _emb_body(x_hbm, table_hbm, out_hbm, idx_v, rows_v, trans_v, isem, gsem, ssem):
    wid = lax.axis_index("s") * 2 + lax.axis_index("c")

    def item(m):
        return m // KPC, lax.rem(m, KPC)

    def idxload(m, buf):
        c, k = item(m)
        pltpu.async_copy(
            x_hbm.at[c, pl.ds(k * CHUNK, CHUNK)], idx_v.at[buf], isem
        )

    def idxwait(m, buf):
        c, k = item(m)
        pltpu.make_async_copy(
            x_hbm.at[c, pl.ds(k * CHUNK, CHUNK)], idx_v.at[buf], isem
        ).wait()

    def gather(buf):
        pltpu.async_copy(table_hbm.at[idx_v.at[buf]], rows_v.at[buf], gsem)

    def gatherwait(buf):
        pltpu.make_async_copy(
            table_hbm.at[idx_v.at[buf]], rows_v.at[buf], gsem
        ).wait()

    def outdma(m, buf):
        c, k = item(m)
        pltpu.async_copy(
            trans_v.at[buf],
            out_hbm.at[c, pl.ds(0, DIM), pl.ds(k * CHUNK, CHUNK)],
            ssem,
        )

    def outwait(m, buf):
        c, k = item(m)
        pltpu.make_async_copy(
            trans_v.at[buf],
            out_hbm.at[c, pl.ds(0, DIM), pl.ds(k * CHUNK, CHUNK)],
            ssem,
        ).wait()

    d_lo = lax.iota(jnp.int32, LANES)
    d_hi = d_lo + LANES

    def transpose(B):
        rows = rows_v.at[B]
        trans = trans_v.at[B]

        def tbody(r, _):
            r_vec = jnp.broadcast_to(r, (LANES,)).astype(jnp.int32)
            lo = plsc.load_gather(rows, [r_vec, d_lo])
            hi = plsc.load_gather(rows, [r_vec, d_hi])
            plsc.store_scatter(trans, [d_lo, r_vec], lo)
            plsc.store_scatter(trans, [d_hi, r_vec], hi)
            return 0

        lax.fori_loop(0, CHUNK, tbody, 0)

    base = wid * IPW
    idxload(base, 0)
    idxwait(base, 0)
    gather(0)

    def step(t, B):
        m = base + t
        gatherwait(B)

        @pl.when(t + 1 < IPW)
        def _():
            idxload(m + 1, 1 - B)
            idxwait(m + 1, 1 - B)
            gather(1 - B)

        @pl.when(t >= 2)
        def _():
            outwait(m - 2, B)

        transpose(B)
        outdma(m, B)

    def body(tt, _):
        step(2 * tt, 0)
        step(2 * tt + 1, 1)
        return 0

    lax.fori_loop(0, IPW // 2, body, 0)
    outwait(base + IPW - 2, 0)
    outwait(base + IPW - 1, 1)


@jax.jit
def _embedding_lookup(xt, table):
    mesh = plsc.VectorSubcoreMesh(core_axis_name="c", subcore_axis_name="s")
    run = functools.partial(
        pl.kernel,
        mesh=mesh,
        compiler_params=pltpu.CompilerParams(
            use_tc_tiling_on_sc=False, needs_layout_passes=False
        ),
        out_type=jax.ShapeDtypeStruct((NUM_COLS, DIM, NUM_ROWS), jnp.float32),
        scratch_types=[
            pltpu.VMEM((2, CHUNK), jnp.int32),
            pltpu.VMEM((2, CHUNK, DIM), jnp.float32),
            pltpu.VMEM((2, DIM, CHUNK), jnp.float32),
            pltpu.SemaphoreType.DMA,
            pltpu.SemaphoreType.DMA,
            pltpu.SemaphoreType.DMA,
        ],
    )(_emb_body)
    return run(xt, table)


def kernel(x, table):
    t128 = jax.lax.optimization_barrier(table.reshape(NUM_EMB // 4, DIM * 4))
    out = _embedding_lookup(x.T, t128.reshape(NUM_EMB, DIM))
    return out.transpose(2, 0, 1)


# speedup vs baseline: 1.0624x; 1.0624x over previous
import functools

import jax
import jax.numpy as jnp
from jax import lax
from jax.experimental import pallas as pl
from jax.experimental.pallas import tpu as pltpu
from jax.experimental.pallas import tpu_sc as plsc

NUM_ROWS = 16384
NUM_COLS = 26
DIM = 32
NUM_EMB = 1000000
B_TOTAL = NUM_ROWS * NUM_COLS
NW = 32
B_PER_W = B_TOTAL // NW
CHUNK = 1024
NCHUNK = B_PER_W // CHUNK


def _emb_body(x_hbm, table_hbm, out_hbm, idx_v, rows_v, gsem, ssem):
    wid = lax.axis_index("s") * 2 + lax.axis_index("c")
    base = wid * B_PER_W
    pltpu.sync_copy(x_hbm.at[pl.ds(base, B_PER_W)], idx_v)

    def gather(g, buf):
        pltpu.async_copy(
            table_hbm.at[idx_v.at[pl.ds(g * CHUNK, CHUNK)]], rows_v.at[buf], gsem
        )

    def store(g, buf):
        pltpu.async_copy(
            rows_v.at[buf], out_hbm.at[pl.ds(base + g * CHUNK, CHUNK)], ssem
        )

    gather(0, 0)

    def body(g, _):
        buf = lax.rem(g, 2)
        nbuf = lax.rem(g + 1, 2)

        @pl.when(g + 1 < NCHUNK)
        def _():
            gather(g + 1, nbuf)

        pltpu.make_async_copy(
            table_hbm.at[idx_v.at[pl.ds(g * CHUNK, CHUNK)]], rows_v.at[buf], gsem
        ).wait()

        @pl.when(g >= 2)
        def _():
            pltpu.make_async_copy(
                rows_v.at[buf], out_hbm.at[pl.ds(base + (g - 2) * CHUNK, CHUNK)], ssem
            ).wait()

        store(g, buf)
        return 0

    lax.fori_loop(0, NCHUNK, body, 0)
    pltpu.make_async_copy(
        rows_v.at[(NCHUNK - 2) % 2],
        out_hbm.at[pl.ds(base + (NCHUNK - 2) * CHUNK, CHUNK)],
        ssem,
    ).wait()
    pltpu.make_async_copy(
        rows_v.at[(NCHUNK - 1) % 2],
        out_hbm.at[pl.ds(base + (NCHUNK - 1) * CHUNK, CHUNK)],
        ssem,
    ).wait()


@jax.jit
def _embedding_lookup(x_flat, table):
    mesh = plsc.VectorSubcoreMesh(core_axis_name="c", subcore_axis_name="s")
    run = functools.partial(
        pl.kernel,
        mesh=mesh,
        compiler_params=pltpu.CompilerParams(use_tc_tiling_on_sc=False),
        out_type=jax.ShapeDtypeStruct((B_TOTAL, DIM), jnp.float32),
        scratch_types=[
            pltpu.VMEM((B_PER_W,), jnp.int32),
            pltpu.VMEM((2, CHUNK, DIM), jnp.float32),
            pltpu.SemaphoreType.DMA,
            pltpu.SemaphoreType.DMA,
        ],
    )(_emb_body)
    return run(x_flat, table)


def kernel(x, table):
    t128 = jax.lax.optimization_barrier(table.reshape(NUM_EMB // 4, DIM * 4))
    out = _embedding_lookup(x.reshape(-1), t128.reshape(NUM_EMB, DIM))
    out128 = jax.lax.optimization_barrier(
        out.reshape(B_TOTAL * DIM // 128, 128)
    )
    return out128.reshape(NUM_ROWS, NUM_COLS, DIM)
